# baseline (speedup 1.0000x reference)
import jax
import jax.numpy as jnp
from jax import lax
from jax.experimental import pallas as pl
from jax.experimental.pallas import tpu as pltpu
from jax.experimental.pallas import tpu_sc as plsc

_H = 32
_N = 64


def _bias_body(rt_hbm, ct_hbm, out_hbm, rt_tab, ct_tab, out_v, sem_r, sem_c,
               sem_o):
    nc = plsc.get_sparse_core_info().num_cores
    wid = lax.axis_index("s") * nc + lax.axis_index("c")

    cp_r = pltpu.async_copy(rt_hbm, rt_tab, sem_r)
    cp_c = pltpu.async_copy(ct_hbm, ct_tab, sem_c)

    lane = lax.broadcasted_iota(jnp.int32, (16,), 0)
    colsel = jnp.full((16,), 0, jnp.int32) + wid

    cp_c.wait()
    cvec = [plsc.load_gather(ct_tab, [p + 7 - (lane & 7), colsel])
            for p in range(8)]

    cp_r.wait()

    copies = []
    for a in range(8):
        rvec = [
            plsc.load_gather(rt_tab, [a + 7 - ((c * 16 + lane) >> 3), colsel])
            for c in range(4)
        ]
        for p in range(8):
            for c in range(4):
                out_v[a * 8 + p, pl.ds(c * 16, 16)] = rvec[c] + cvec[p]
        copies.append(pltpu.async_copy(
            out_v.at[pl.ds(a * 8, 8)],
            out_hbm.at[0, wid, pl.ds(a * 8, 8)],
            sem_o,
        ))
    for cp in copies:
        cp.wait()


@jax.jit
def _bias_planes(row_table, col_table):
    mesh = plsc.VectorSubcoreMesh(core_axis_name="c", subcore_axis_name="s")
    return pl.kernel(
        _bias_body,
        mesh=mesh,
        out_type=jax.ShapeDtypeStruct((1, _H, _N, _N), jnp.float32),
        scratch_types=[
            pltpu.VMEM((15, _H), jnp.float32),
            pltpu.VMEM((15, _H), jnp.float32),
            pltpu.VMEM((_N, _N), jnp.float32),
            pltpu.SemaphoreType.DMA,
            pltpu.SemaphoreType.DMA,
            pltpu.SemaphoreType.DMA,
        ],
        compiler_params=pltpu.CompilerParams(
            needs_layout_passes=False,
            disable_bounds_checks=True,
            skip_device_barrier=True,
        ),
    )(row_table, col_table)


def kernel(q_len, k_len, row_bias_table, col_bias_table):
    return _bias_planes(row_bias_table, col_bias_table)


# speedup vs baseline: 1.0797x; 1.0797x over previous
import jax
import jax.numpy as jnp
from jax import lax
from jax.experimental import pallas as pl
from jax.experimental.pallas import tpu as pltpu
from jax.experimental.pallas import tpu_sc as plsc

_H = 32
_N = 64


def _bias_body(rt_hbm, ct_hbm, out_hbm, rt_tab, ct_tab, out_v, sem_r, sem_c):
    wid = lax.axis_index("s")

    cp_r = pltpu.async_copy(rt_hbm, rt_tab, sem_r)
    cp_c = pltpu.async_copy(ct_hbm, ct_tab, sem_c)

    lane = lax.broadcasted_iota(jnp.int32, (16,), 0)

    cp_c.wait()
    cp_r.wait()

    for hh in range(2):
        colsel = jnp.full((16,), hh, jnp.int32) + 2 * wid

        cvec = [plsc.load_gather(ct_tab, [p + 7 - (lane & 7), colsel])
                for p in range(8)]

        def block_body(a, _):
            rvec = [
                plsc.load_gather(
                    rt_tab, [a + 7 - ((c * 16 + lane) >> 3), colsel])
                for c in range(4)
            ]
            for p in range(8):
                for c in range(4):
                    out_v[hh, a * 8 + p, pl.ds(c * 16, 16)] = rvec[c] + cvec[p]
            return 0

        lax.fori_loop(0, 8, block_body, 0)

    pltpu.sync_copy(out_v, out_hbm.at[0, pl.ds(2 * wid, 2)])


@jax.jit
def _bias_planes(row_table, col_table):
    mesh = plsc.VectorSubcoreMesh(
        core_axis_name="c", subcore_axis_name="s", num_cores=1)
    return pl.kernel(
        _bias_body,
        mesh=mesh,
        out_type=jax.ShapeDtypeStruct((1, _H, _N, _N), jnp.float32),
        scratch_types=[
            pltpu.VMEM((15, _H), jnp.float32),
            pltpu.VMEM((15, _H), jnp.float32),
            pltpu.VMEM((2, _N, _N), jnp.float32),
            pltpu.SemaphoreType.DMA,
            pltpu.SemaphoreType.DMA,
        ],
        compiler_params=pltpu.CompilerParams(
            needs_layout_passes=False,
            disable_bounds_checks=True,
            skip_device_barrier=True,
        ),
    )(row_table, col_table)


def kernel(q_len, k_len, row_bias_table, col_bias_table):
    return _bias_planes(row_bias_table, col_bias_table)


# speedup vs baseline: 1.0891x; 1.0087x over previous
import jax
import jax.numpy as jnp
from jax import lax
from jax.experimental import pallas as pl
from jax.experimental.pallas import tpu as pltpu
from jax.experimental.pallas import tpu_sc as plsc

_H = 32
_N = 64


def _bias_body(rt_hbm, ct_hbm, out_hbm, rt_tab, ct_tab, out_v, sem_r, sem_c):
    wid = lax.axis_index("s")

    cp_r = pltpu.async_copy(rt_hbm, rt_tab, sem_r)
    cp_c = pltpu.async_copy(ct_hbm, ct_tab, sem_c)

    lane = lax.broadcasted_iota(jnp.int32, (16,), 0)

    cp_c.wait()
    cp_r.wait()

    def head_body(hh, _):
        colsel = jnp.full((16,), 0, jnp.int32) + (2 * wid + hh)

        cvec = [plsc.load_gather(ct_tab, [p + 7 - (lane & 7), colsel])
                for p in range(8)]

        def block_body(a, _):
            rvec = [
                plsc.load_gather(
                    rt_tab, [a + 7 - ((c * 16 + lane) >> 3), colsel])
                for c in range(4)
            ]
            for p in range(8):
                for c in range(4):
                    out_v[hh, a * 8 + p, pl.ds(c * 16, 16)] = rvec[c] + cvec[p]
            return 0

        lax.fori_loop(0, 8, block_body, 0)
        return 0

    lax.fori_loop(0, 2, head_body, 0)

    pltpu.sync_copy(out_v, out_hbm.at[0, pl.ds(2 * wid, 2)])


@jax.jit
def _bias_planes(row_table, col_table):
    mesh = plsc.VectorSubcoreMesh(
        core_axis_name="c", subcore_axis_name="s", num_cores=1)
    return pl.kernel(
        _bias_body,
        mesh=mesh,
        out_type=jax.ShapeDtypeStruct((1, _H, _N, _N), jnp.float32),
        scratch_types=[
            pltpu.VMEM((15, _H), jnp.float32),
            pltpu.VMEM((15, _H), jnp.float32),
            pltpu.VMEM((2, _N, _N), jnp.float32),
            pltpu.SemaphoreType.DMA,
            pltpu.SemaphoreType.DMA,
        ],
        compiler_params=pltpu.CompilerParams(
            needs_layout_passes=False,
            disable_bounds_checks=True,
            skip_device_barrier=True,
        ),
    )(row_table, col_table)


def kernel(q_len, k_len, row_bias_table, col_bias_table):
    return _bias_planes(row_bias_table, col_bias_table)


# speedup vs baseline: 1.0908x; 1.0016x over previous
import jax
import jax.numpy as jnp
from jax import lax
from jax.experimental import pallas as pl
from jax.experimental.pallas import tpu as pltpu
from jax.experimental.pallas import tpu_sc as plsc

_H = 32
_N = 64


def _bias_body(rt_hbm, ct_hbm, out_hbm, rt_tab, ct_tab, out_v, sem_r, sem_c,
               sem_o):
    wid = lax.axis_index("s")

    cp_r = pltpu.async_copy(rt_hbm, rt_tab, sem_r)
    cp_c = pltpu.async_copy(ct_hbm, ct_tab, sem_c)

    lane = lax.broadcasted_iota(jnp.int32, (16,), 0)

    cp_c.wait()
    cp_r.wait()

    def build_plane(hh):
        colsel = jnp.full((16,), hh, jnp.int32) + 2 * wid

        cvec = [plsc.load_gather(ct_tab, [p + 7 - (lane & 7), colsel])
                for p in range(8)]

        def block_body(a, _):
            rvec = [
                plsc.load_gather(
                    rt_tab, [a + 7 - ((c * 16 + lane) >> 3), colsel])
                for c in range(4)
            ]
            for p in range(8):
                for c in range(4):
                    out_v[hh, a * 8 + p, pl.ds(c * 16, 16)] = rvec[c] + cvec[p]
            return 0

        lax.fori_loop(0, 8, block_body, 0)

    build_plane(0)
    cp0 = pltpu.async_copy(out_v.at[0], out_hbm.at[0, 2 * wid], sem_o)
    build_plane(1)
    cp1 = pltpu.async_copy(out_v.at[1], out_hbm.at[0, 2 * wid + 1], sem_o)
    cp0.wait()
    cp1.wait()


@jax.jit
def _bias_planes(row_table, col_table):
    mesh = plsc.VectorSubcoreMesh(
        core_axis_name="c", subcore_axis_name="s", num_cores=1)
    return pl.kernel(
        _bias_body,
        mesh=mesh,
        out_type=jax.ShapeDtypeStruct((1, _H, _N, _N), jnp.float32),
        scratch_types=[
            pltpu.VMEM((15, _H), jnp.float32),
            pltpu.VMEM((15, _H), jnp.float32),
            pltpu.VMEM((2, _N, _N), jnp.float32),
            pltpu.SemaphoreType.DMA,
            pltpu.SemaphoreType.DMA,
            pltpu.SemaphoreType.DMA,
        ],
        compiler_params=pltpu.CompilerParams(
            needs_layout_passes=False,
            disable_bounds_checks=True,
            skip_device_barrier=True,
        ),
    )(row_table, col_table)


def kernel(q_len, k_len, row_bias_table, col_bias_table):
    return _bias_planes(row_bias_table, col_bias_table)


# speedup vs baseline: 1.1029x; 1.0111x over previous
import jax
import jax.numpy as jnp
from jax import lax
from jax.experimental import pallas as pl
from jax.experimental.pallas import tpu as pltpu
from jax.experimental.pallas import tpu_sc as plsc

_H = 32
_N = 64


def _bias_body(tabs_hbm, out_hbm, tabs_v, out_v, sem_t, sem_o):
    wid = lax.axis_index("s")

    cp_t = pltpu.async_copy(tabs_hbm, tabs_v, sem_t)

    lane = lax.broadcasted_iota(jnp.int32, (16,), 0)

    cp_t.wait()
    rt_tab = tabs_v.at[0]
    ct_tab = tabs_v.at[1]

    def build_plane(hh):
        colsel = jnp.full((16,), hh, jnp.int32) + 2 * wid

        cvec = [plsc.load_gather(ct_tab, [p + 7 - (lane & 7), colsel])
                for p in range(8)]

        def block_body(a, _):
            rvec = [
                plsc.load_gather(
                    rt_tab, [a + 7 - ((c * 16 + lane) >> 3), colsel])
                for c in range(4)
            ]
            for p in range(8):
                for c in range(4):
                    out_v[hh, a * 8 + p, pl.ds(c * 16, 16)] = rvec[c] + cvec[p]
            return 0

        lax.fori_loop(0, 8, block_body, 0)

    build_plane(0)
    cp0 = pltpu.async_copy(out_v.at[0], out_hbm.at[0, 2 * wid], sem_o)
    build_plane(1)
    cp1 = pltpu.async_copy(out_v.at[1], out_hbm.at[0, 2 * wid + 1], sem_o)
    cp0.wait()
    cp1.wait()


@jax.jit
def _bias_planes(row_table, col_table):
    tabs = jnp.stack([row_table, col_table])
    mesh = plsc.VectorSubcoreMesh(
        core_axis_name="c", subcore_axis_name="s", num_cores=1)
    return pl.kernel(
        _bias_body,
        mesh=mesh,
        out_type=jax.ShapeDtypeStruct((1, _H, _N, _N), jnp.float32),
        scratch_types=[
            pltpu.VMEM((2, 15, _H), jnp.float32),
            pltpu.VMEM((2, _N, _N), jnp.float32),
            pltpu.SemaphoreType.DMA,
            pltpu.SemaphoreType.DMA,
        ],
        compiler_params=pltpu.CompilerParams(
            needs_layout_passes=False,
            disable_bounds_checks=True,
            skip_device_barrier=True,
        ),
    )(tabs)


def kernel(q_len, k_len, row_bias_table, col_bias_table):
    return _bias_planes(row_bias_table, col_bias_table)


# speedup vs baseline: 1.1093x; 1.0057x over previous
import jax
import jax.numpy as jnp
from jax import lax
from jax.experimental import pallas as pl
from jax.experimental.pallas import tpu as pltpu
from jax.experimental.pallas import tpu_sc as plsc

_H = 32
_N = 64


def _bias_body(tabs_hbm, out_hbm, tabs_v, out_v, sem_t, sem_o):
    wid = lax.axis_index("s")

    cp_t = pltpu.async_copy(tabs_hbm, tabs_v, sem_t)

    lane = lax.broadcasted_iota(jnp.int32, (16,), 0)

    cp_t.wait()
    rt_tab = tabs_v.at[0]
    ct_tab = tabs_v.at[1]

    def build_plane(hh):
        colsel = jnp.full((16,), hh, jnp.int32) + 2 * wid

        cvec = [plsc.load_gather(ct_tab, [p + 7 - (lane & 7), colsel])
                for p in range(8)]

        @plsc.parallel_loop(0, 8, unroll=2)
        def block_body(a):
            rvec = [
                plsc.load_gather(
                    rt_tab, [a + 7 - ((c * 16 + lane) >> 3), colsel])
                for c in range(4)
            ]
            for p in range(8):
                for c in range(4):
                    out_v[hh, a * 8 + p, pl.ds(c * 16, 16)] = rvec[c] + cvec[p]

    build_plane(0)
    cp0 = pltpu.async_copy(out_v.at[0], out_hbm.at[0, 2 * wid], sem_o)
    build_plane(1)
    cp1 = pltpu.async_copy(out_v.at[1], out_hbm.at[0, 2 * wid + 1], sem_o)
    cp0.wait()
    cp1.wait()


@jax.jit
def _bias_planes(row_table, col_table):
    tabs = jnp.stack([row_table, col_table])
    mesh = plsc.VectorSubcoreMesh(
        core_axis_name="c", subcore_axis_name="s", num_cores=1)
    return pl.kernel(
        _bias_body,
        mesh=mesh,
        out_type=jax.ShapeDtypeStruct((1, _H, _N, _N), jnp.float32),
        scratch_types=[
            pltpu.VMEM((2, 15, _H), jnp.float32),
            pltpu.VMEM((2, _N, _N), jnp.float32),
            pltpu.SemaphoreType.DMA,
            pltpu.SemaphoreType.DMA,
        ],
        compiler_params=pltpu.CompilerParams(
            needs_layout_passes=False,
            disable_bounds_checks=True,
            skip_device_barrier=True,
        ),
    )(tabs)


def kernel(q_len, k_len, row_bias_table, col_bias_table):
    return _bias_planes(row_bias_table, col_bias_table)
